# baseline (speedup 1.0000x reference)
import functools

import jax
import jax.numpy as jnp
from jax import lax
from jax.experimental import pallas as pl
from jax.experimental.pallas import tpu as pltpu

_EPS = float(jnp.finfo(jnp.float32).eps)
_S = 16
_RC = 1536


def _cls_body(nsteps, *refs):
    yt_refs = refs[:_S]
    yp_refs = refs[_S:2 * _S]
    pp_out, tp_out, ppa_ref, tpa_ref = refs[2 * _S:]
    i = pl.program_id(0)

    @pl.when(i == 0)
    def _init():
        ppa_ref[...] = jnp.zeros_like(ppa_ref)
        tpa_ref[...] = jnp.zeros_like(tpa_ref)

    for yt_ref, yp_ref in zip(yt_refs, yp_refs):
        xt = yt_ref[...]
        xp = yp_ref[...]
        mt = jnp.max(xt, axis=0, keepdims=True)
        mp = jnp.max(xp, axis=0, keepdims=True)
        eq_t = xt == mt
        eq_p = xp == mp
        ppf = eq_p.astype(jnp.float32)
        tpf = (eq_t & eq_p).astype(jnp.float32)
        C, RC = xt.shape
        g = RC // 128
        accp = ppf[:, 0:128]
        acct = tpf[:, 0:128]
        for j in range(1, g):
            accp = accp + ppf[:, j * 128:(j + 1) * 128]
            acct = acct + tpf[:, j * 128:(j + 1) * 128]
        ppa_ref[...] += accp
        tpa_ref[...] += acct

    @pl.when(i == nsteps - 1)
    def _fin():
        pp_out[...] = jnp.sum(ppa_ref[...], axis=1, keepdims=True)
        tp_out[...] = jnp.sum(tpa_ref[...], axis=1, keepdims=True)


def _fin_body(ytr_ref, ypr_ref, ppm_ref, tpm_ref, out_ref):
    xt = ytr_ref[...]
    xp = ypr_ref[...]
    mt = jnp.max(xt, axis=0, keepdims=True)
    mp = jnp.max(xp, axis=0, keepdims=True)
    eq_t = xt == mt
    eq_p = xp == mp
    pp = ppm_ref[...] + jnp.sum(eq_p.astype(jnp.float32), axis=1, keepdims=True)
    tp = tpm_ref[...] + jnp.sum((eq_t & eq_p).astype(jnp.float32), axis=1,
                                keepdims=True)
    C = pp.shape[0]
    prec = tp / (pp + _EPS)
    out_ref[...] = jnp.sum(prec, axis=0, keepdims=True) / jnp.float32(C)


def _spec(C, s, G):
    return pl.BlockSpec((C, _RC), lambda i, _s=s, _G=G: (0, i + _s * _G))


def kernel(y_true, y_pred):
    N, C = y_true.shape
    ytT = y_true.T
    ypT = y_pred.T
    G = N // (_S * _RC)
    nmain = _S * G * _RC
    in_specs = ([_spec(C, s, G) for s in range(_S)] +
                [_spec(C, s, G) for s in range(_S)])
    pp_m, tp_m = pl.pallas_call(
        functools.partial(_cls_body, G),
        grid=(G,),
        in_specs=in_specs,
        out_specs=[
            pl.BlockSpec((C, 1), lambda i: (0, 0)),
            pl.BlockSpec((C, 1), lambda i: (0, 0)),
        ],
        out_shape=[
            jax.ShapeDtypeStruct((C, 1), jnp.float32),
            jax.ShapeDtypeStruct((C, 1), jnp.float32),
        ],
        scratch_shapes=[
            pltpu.VMEM((C, 128), jnp.float32),
            pltpu.VMEM((C, 128), jnp.float32),
        ],
        compiler_params=pltpu.CompilerParams(dimension_semantics=("arbitrary",)),
    )(*([ytT] * _S + [ypT] * _S))

    ytR = lax.slice(ytT, (0, nmain), (C, N))
    ypR = lax.slice(ypT, (0, nmain), (C, N))
    out = pl.pallas_call(
        _fin_body,
        out_shape=jax.ShapeDtypeStruct((1, 1), jnp.float32),
    )(ytR, ypR, pp_m, tp_m)
    return out[0, 0]


# speedup vs baseline: 1.0276x; 1.0276x over previous
import functools

import jax
import jax.numpy as jnp
from jax import lax
from jax.experimental import pallas as pl
from jax.experimental.pallas import tpu as pltpu

_EPS = float(jnp.finfo(jnp.float32).eps)
_S = 8
_RC = 3456


def _cls_body(nsteps, *refs):
    yt_refs = refs[:_S]
    yp_refs = refs[_S:2 * _S]
    pp_out, tp_out, ppa_ref, tpa_ref = refs[2 * _S:]
    i = pl.program_id(0)

    @pl.when(i == 0)
    def _init():
        ppa_ref[...] = jnp.zeros_like(ppa_ref)
        tpa_ref[...] = jnp.zeros_like(tpa_ref)

    for yt_ref, yp_ref in zip(yt_refs, yp_refs):
        xt = yt_ref[...]
        xp = yp_ref[...]
        mt = jnp.max(xt, axis=0, keepdims=True)
        mp = jnp.max(xp, axis=0, keepdims=True)
        eq_t = xt == mt
        eq_p = xp == mp
        ppf = eq_p.astype(jnp.float32)
        tpf = (eq_t & eq_p).astype(jnp.float32)
        C, RC = xt.shape
        g = RC // 128
        accp = ppf[:, 0:128]
        acct = tpf[:, 0:128]
        for j in range(1, g):
            accp = accp + ppf[:, j * 128:(j + 1) * 128]
            acct = acct + tpf[:, j * 128:(j + 1) * 128]
        ppa_ref[...] += accp
        tpa_ref[...] += acct

    @pl.when(i == nsteps - 1)
    def _fin():
        pp_out[...] = jnp.sum(ppa_ref[...], axis=1, keepdims=True)
        tp_out[...] = jnp.sum(tpa_ref[...], axis=1, keepdims=True)


def _fin_body(ytr_ref, ypr_ref, ppm_ref, tpm_ref, out_ref):
    xt = ytr_ref[...]
    xp = ypr_ref[...]
    mt = jnp.max(xt, axis=0, keepdims=True)
    mp = jnp.max(xp, axis=0, keepdims=True)
    eq_t = xt == mt
    eq_p = xp == mp
    pp = ppm_ref[...] + jnp.sum(eq_p.astype(jnp.float32), axis=1, keepdims=True)
    tp = tpm_ref[...] + jnp.sum((eq_t & eq_p).astype(jnp.float32), axis=1,
                                keepdims=True)
    C = pp.shape[0]
    prec = tp / (pp + _EPS)
    out_ref[...] = jnp.sum(prec, axis=0, keepdims=True) / jnp.float32(C)


def _spec(C, s, G):
    return pl.BlockSpec((C, _RC), lambda i, _s=s, _G=G: (0, i + _s * _G))


def kernel(y_true, y_pred):
    N, C = y_true.shape
    ytT = y_true.T
    ypT = y_pred.T
    G = N // (_S * _RC)
    nmain = _S * G * _RC
    in_specs = ([_spec(C, s, G) for s in range(_S)] +
                [_spec(C, s, G) for s in range(_S)])
    pp_m, tp_m = pl.pallas_call(
        functools.partial(_cls_body, G),
        grid=(G,),
        in_specs=in_specs,
        out_specs=[
            pl.BlockSpec((C, 1), lambda i: (0, 0)),
            pl.BlockSpec((C, 1), lambda i: (0, 0)),
        ],
        out_shape=[
            jax.ShapeDtypeStruct((C, 1), jnp.float32),
            jax.ShapeDtypeStruct((C, 1), jnp.float32),
        ],
        scratch_shapes=[
            pltpu.VMEM((C, 128), jnp.float32),
            pltpu.VMEM((C, 128), jnp.float32),
        ],
        compiler_params=pltpu.CompilerParams(dimension_semantics=("arbitrary",)),
    )(*([ytT] * _S + [ypT] * _S))

    ytR = lax.slice(ytT, (0, nmain), (C, N))
    ypR = lax.slice(ypT, (0, nmain), (C, N))
    out = pl.pallas_call(
        _fin_body,
        out_shape=jax.ShapeDtypeStruct((1, 1), jnp.float32),
    )(ytR, ypR, pp_m, tp_m)
    return out[0, 0]
